# baseline (speedup 1.0000x reference)
import jax
import jax.numpy as jnp
from jax.experimental import pallas as pl
from jax.experimental.pallas import tpu as pltpu

D_MODEL = 768
NUM_EXPERTS = 64
TOP_K = 2

_BLOCK = 4096
_CHUNK = 1024


def _router_block(x_ref, w_ref, b_ref, logits_ref, row8_ref):
    w = w_ref[...]
    b = b_ref[...]
    for c in range(_BLOCK // _CHUNK):
        lo = c * _CHUNK
        logits = jax.lax.dot_general(
            x_ref[lo:lo + _CHUNK, :], w,
            dimension_numbers=(((1,), (1,)), ((), ())),
            preferred_element_type=jnp.float32)
        logits = logits + b
        logits_ref[0, lo:lo + _CHUNK, :] = logits

        iota = jax.lax.broadcasted_iota(
            jnp.int32, logits.shape, 1).astype(jnp.float32)
        m1 = jnp.max(logits, axis=1, keepdims=True)
        i1 = jnp.min(jnp.where(logits == m1, iota, jnp.float32(NUM_EXPERTS)),
                     axis=1, keepdims=True)
        masked = jnp.where(iota == i1, -jnp.inf, logits)
        m2 = jnp.max(masked, axis=1, keepdims=True)
        i2 = jnp.min(jnp.where(masked == m2, iota, jnp.float32(NUM_EXPERTS)),
                     axis=1, keepdims=True)
        e2 = jnp.exp(m2 - m1)
        g1 = 1.0 / (1.0 + e2)
        cols = jnp.concatenate([i1, i2, g1, e2 * g1, i1, i2, g1, e2 * g1],
                               axis=1)
        row8_ref[:, lo:lo + _CHUNK] = jnp.transpose(cols, (1, 0))


@jax.jit
def kernel(x, W, b):
    B, S, D = x.shape
    n = B * S
    x2 = x.reshape(n, D)
    Wt = W.T
    b2 = b.reshape(1, NUM_EXPERTS)

    nh = n // 2
    Bh = B // 2
    sb = S // _BLOCK
    halves = []
    for h in range(2):
        base = h * (nh // _BLOCK)
        halves.append(pl.pallas_call(
            _router_block,
            grid=(nh // _BLOCK,),
            in_specs=[
                pl.BlockSpec((_BLOCK, D), lambda i, base=base: (base + i, 0)),
                pl.BlockSpec((NUM_EXPERTS, D), lambda i: (0, 0)),
                pl.BlockSpec((1, NUM_EXPERTS), lambda i: (0, 0)),
            ],
            out_specs=[
                pl.BlockSpec((1, _BLOCK, NUM_EXPERTS),
                             lambda i: (i // sb, i % sb, 0)),
                pl.BlockSpec((8, _BLOCK), lambda i: (0, i)),
            ],
            out_shape=[
                jax.ShapeDtypeStruct((Bh, S, NUM_EXPERTS), jnp.float32),
                jax.ShapeDtypeStruct((8, nh), jnp.float32),
            ],
            compiler_params=pltpu.CompilerParams(
                dimension_semantics=("parallel",),
                vmem_limit_bytes=100 * 1024 * 1024,
            ),
        )(x2, Wt, b2))
    logits = jnp.concatenate([lg for lg, _ in halves], axis=0)
    row8 = jnp.concatenate([r8 for _, r8 in halves], axis=1)

    idx = jnp.transpose(row8[:TOP_K, :], (1, 0)).astype(jnp.int32)
    gates = jnp.transpose(row8[TOP_K:2 * TOP_K, :], (1, 0))
    return (logits,
            idx.reshape(B, S, TOP_K),
            gates.reshape(B, S, TOP_K))


# speedup vs baseline: 1.4464x; 1.4464x over previous
import jax
import jax.numpy as jnp
from jax.experimental import pallas as pl
from jax.experimental.pallas import tpu as pltpu

D_MODEL = 768
NUM_EXPERTS = 64
TOP_K = 2

_BLOCK = 4096
_CHUNK = 1024


def _router_block(x_ref, w_ref, b_ref, logits_ref, row8_ref):
    w = w_ref[...]
    b = b_ref[...]
    for c in range(_BLOCK // _CHUNK):
        lo = c * _CHUNK
        logits = jax.lax.dot_general(
            x_ref[lo:lo + _CHUNK, :], w,
            dimension_numbers=(((1,), (1,)), ((), ())),
            preferred_element_type=jnp.float32)
        logits = logits + b
        logits_ref[0, lo:lo + _CHUNK, :] = logits

        iota = jax.lax.broadcasted_iota(
            jnp.int32, logits.shape, 1).astype(jnp.float32)
        m1 = jnp.max(logits, axis=1, keepdims=True)
        i1 = jnp.min(jnp.where(logits == m1, iota, jnp.float32(NUM_EXPERTS)),
                     axis=1, keepdims=True)
        masked = jnp.where(iota == i1, -jnp.inf, logits)
        m2 = jnp.max(masked, axis=1, keepdims=True)
        i2 = jnp.min(jnp.where(masked == m2, iota, jnp.float32(NUM_EXPERTS)),
                     axis=1, keepdims=True)
        e2 = jnp.exp(m2 - m1)
        g1 = 1.0 / (1.0 + e2)
        cols = jnp.concatenate([i1, i2, g1, e2 * g1, i1, i2, g1, e2 * g1],
                               axis=1)
        row8_ref[:, lo:lo + _CHUNK] = jnp.transpose(cols, (1, 0))


@jax.jit
def kernel(x, W, b):
    B, S, D = x.shape
    n = B * S
    x2 = x.reshape(n, D)
    Wt = W.T
    b2 = b.reshape(1, NUM_EXPERTS)

    grid = (n // _BLOCK,)
    logits, row8 = pl.pallas_call(
        _router_block,
        grid=grid,
        in_specs=[
            pl.BlockSpec((_BLOCK, D), lambda i: (i, 0)),
            pl.BlockSpec((NUM_EXPERTS, D), lambda i: (0, 0)),
            pl.BlockSpec((1, NUM_EXPERTS), lambda i: (0, 0)),
        ],
        out_specs=[
            pl.BlockSpec((1, _BLOCK, NUM_EXPERTS),
                         lambda i: (i // (S // _BLOCK), i % (S // _BLOCK), 0)),
            pl.BlockSpec((8, _BLOCK), lambda i: (0, i)),
        ],
        out_shape=[
            jax.ShapeDtypeStruct((B, S, NUM_EXPERTS), jnp.float32),
            jax.ShapeDtypeStruct((8, n), jnp.float32),
        ],
        compiler_params=pltpu.CompilerParams(
            dimension_semantics=("parallel",),
            vmem_limit_bytes=100 * 1024 * 1024,
        ),
    )(x2, Wt, b2)

    idx = jnp.transpose(row8[:TOP_K, :], (1, 0)).astype(jnp.int32)
    gates = jnp.transpose(row8[TOP_K:2 * TOP_K, :], (1, 0))
    return (logits,
            idx.reshape(B, S, TOP_K),
            gates.reshape(B, S, TOP_K))
